# baseline (speedup 1.0000x reference)
import functools

import jax
import jax.numpy as jnp
from jax import lax
from jax.experimental import pallas as pl
from jax.experimental.pallas import tpu as pltpu
from jax.experimental.pallas import tpu_sc as plsc

_B = 16384
_D = 32
_LANES = 16

_NC = 2
_NS = 16
_NW = _NC * _NS
_BPW = _B // _NW


def _sc_body(uidx_hbm, iidx_hbm, utab_hbm, itab_hbm, out_hbm,
             uidx_v, iidx_v, urows_v, irows_v, out_v, sem):
    wid = lax.axis_index("s") * _NC + lax.axis_index("c")
    base = wid * _BPW

    pltpu.sync_copy(uidx_hbm.at[pl.ds(base, _BPW)], uidx_v)
    pltpu.sync_copy(iidx_hbm.at[pl.ds(base, _BPW)], iidx_v)
    cu = pltpu.async_copy(utab_hbm.at[uidx_v], urows_v, sem)
    ci = pltpu.async_copy(itab_hbm.at[iidx_v], irows_v, sem)
    cu.wait()
    ci.wait()

    lane_iota = lax.iota(jnp.int32, _LANES)

    def group(g, carry):
        gbase = g * _LANES
        acc = jnp.zeros((_LANES,), jnp.float32)
        for j in range(_LANES):
            r = gbase + j
            u0 = urows_v[r, pl.ds(0, _LANES)]
            u1 = urows_v[r, pl.ds(_LANES, _LANES)]
            i0 = irows_v[r, pl.ds(0, _LANES)]
            i1 = irows_v[r, pl.ds(_LANES, _LANES)]
            s = jnp.sum(u0 * i0 + u1 * i1)
            acc = jnp.where(lane_iota == j, s, acc)
        out_v[pl.ds(gbase, _LANES)] = 1.0 / (1.0 + jnp.exp(-acc))
        return carry

    lax.fori_loop(0, _BPW // _LANES, group, 0)

    pltpu.sync_copy(out_v, out_hbm.at[pl.ds(base, _BPW)])


@jax.jit
def _mf_forward(user_indices, item_indices, user_table, item_table):
    mesh = plsc.VectorSubcoreMesh(core_axis_name="c", subcore_axis_name="s")
    run = functools.partial(
        pl.kernel,
        mesh=mesh,
        compiler_params=pltpu.CompilerParams(
            needs_layout_passes=False, use_tc_tiling_on_sc=False
        ),
        out_type=jax.ShapeDtypeStruct((_B,), jnp.float32),
        scratch_types=[
            pltpu.VMEM((_BPW,), jnp.int32),
            pltpu.VMEM((_BPW,), jnp.int32),
            pltpu.VMEM((_BPW, _D), jnp.float32),
            pltpu.VMEM((_BPW, _D), jnp.float32),
            pltpu.VMEM((_BPW,), jnp.float32),
            pltpu.SemaphoreType.DMA,
        ],
    )(_sc_body)
    return run(user_indices, item_indices, user_table, item_table)


def kernel(user_indices, item_indices, user_table, item_table):
    return _mf_forward(
        user_indices.astype(jnp.int32),
        item_indices.astype(jnp.int32),
        user_table,
        item_table,
    )
